# Initial kernel scaffold; baseline (speedup 1.0000x reference)
import jax
import jax.numpy as jnp
from jax.experimental import pallas as pl


def kernel(x, edge_index, batch, W_in, b_in, g_in, be_in, W_mid, b_mid, g_mid, be_mid, W_out, b_out, g_out, be_out, W1, b1, W2, b2):
    raise NotImplementedError("write your pallas kernel here")


# speedup vs baseline: 11.2128x; 11.2128x over previous
import functools

import jax
import jax.numpy as jnp
from jax import lax
from jax.experimental import pallas as pl
from jax.experimental.pallas import tpu as pltpu
from jax.experimental.pallas import tpu_sc as plsc

N = 50000
E = 800000
D_IN = 128
D = 64
HALF = 32
G = 32

CHUNK = 128
NSUB = 16
NCORE = 2
EP = 200 * NSUB * NCORE * CHUNK
NROW = EP // CHUNK
ROWS_PER_TILE = NROW // NSUB
ROWS_PER_WORKER = NROW // (NSUB * NCORE)
NP2 = 50048
RPT = NP2 // NSUB
NB = 4

RBLK = 3128
GRID = NP2 // RBLK

_f32 = jnp.float32


def _deg_body(dst2d, zeros8, ones8, deg_out, deg_sh, didx, ones_v, sem):
  c = lax.axis_index("c")
  s = lax.axis_index("s")
  w = s * NCORE + c

  pltpu.sync_copy(ones8, ones_v)

  @pl.when(s == 0)
  def _():
    pltpu.sync_copy(zeros8, deg_sh)

  plsc.subcore_barrier()

  base = w * ROWS_PER_WORKER
  pltpu.sync_copy(dst2d.at[pl.ds(base, ROWS_PER_WORKER)], didx)

  @pl.loop(0, ROWS_PER_WORKER)
  def _(k):
    pltpu.async_copy(ones_v, deg_sh.at[didx.at[k]], sem, add=True)

  @pl.loop(0, ROWS_PER_WORKER)
  def _(k):
    pltpu.make_async_copy(ones_v, deg_sh.at[didx.at[k]], sem).wait()

  plsc.subcore_barrier()
  pltpu.sync_copy(deg_sh.at[pl.ds(s * RPT, RPT)],
                  deg_out.at[c].at[pl.ds(s * RPT, RPT)])


def _deg_call(dst2d, zeros8, ones8):
  mesh = plsc.VectorSubcoreMesh(core_axis_name="c", subcore_axis_name="s")
  return pl.kernel(
      _deg_body,
      out_type=jax.ShapeDtypeStruct((NCORE, NP2, 8), _f32),
      mesh=mesh,
      scratch_types=[
          pltpu.VMEM_SHARED((NP2, 8), _f32),
          pltpu.VMEM((ROWS_PER_WORKER, CHUNK), jnp.int32),
          pltpu.VMEM((CHUNK, 8), _f32),
          pltpu.SemaphoreType.DMA,
      ],
      compiler_params=pltpu.CompilerParams(use_tc_tiling_on_sc=False),
  )(dst2d, zeros8, ones8)


def _agg_body(ei3, hp3, acc_out, acc_sh, ibuf, rowbuf, *sems):
  sem_a = sems[:NB]
  sem_g = sems[NB:2 * NB]
  sem_c = sems[2 * NB:]
  c = lax.axis_index("c")
  s = lax.axis_index("s")

  base = s * ROWS_PER_TILE
  pltpu.sync_copy(hp3.at[c].at[pl.ds(s * RPT, RPT)],
                  acc_sh.at[pl.ds(s * RPT, RPT)])
  plsc.subcore_barrier()

  ngroup = ROWS_PER_TILE // NB

  for b in range(NB):
    pltpu.async_copy(ei3.at[base + b], ibuf.at[b], sem_a[b])

  @pl.loop(0, ngroup)
  def _(g):
    k0 = base + g * NB
    par = (g % 2) * NB
    nxt = ((g + 1) % 2) * NB
    for b in range(NB):
      @pl.when(g > 0)
      def _():
        pltpu.make_async_copy(rowbuf.at[b],
                              acc_sh.at[ibuf.at[nxt + b].at[1]],
                              sem_c[b]).wait()
    for b in range(NB):
      @pl.when(g + 1 < ngroup)
      def _():
        pltpu.async_copy(ei3.at[k0 + NB + b], ibuf.at[nxt + b], sem_a[b])
    for b in range(NB):
      pltpu.make_async_copy(ei3.at[k0 + b], ibuf.at[par + b], sem_a[b]).wait()
      pltpu.async_copy(hp3.at[c].at[ibuf.at[par + b].at[0]], rowbuf.at[b],
                       sem_g[b])
    for b in range(NB):
      pltpu.make_async_copy(hp3.at[c].at[ibuf.at[par + b].at[0]],
                            rowbuf.at[b], sem_g[b]).wait()
      pltpu.async_copy(rowbuf.at[b], acc_sh.at[ibuf.at[par + b].at[1]],
                       sem_c[b], add=True)

  for b in range(NB):
    pltpu.make_async_copy(rowbuf.at[b], acc_sh.at[ibuf.at[b].at[1]],
                          sem_c[b]).wait()

  plsc.subcore_barrier()
  pltpu.sync_copy(acc_sh.at[pl.ds(s * RPT, RPT)],
                  acc_out.at[c].at[pl.ds(s * RPT, RPT)])


def _agg_call(ei3, hp3):
  mesh = plsc.VectorSubcoreMesh(core_axis_name="c", subcore_axis_name="s")
  return pl.kernel(
      _agg_body,
      out_type=jax.ShapeDtypeStruct((NCORE, NP2, HALF), _f32),
      mesh=mesh,
      scratch_types=[
          pltpu.VMEM_SHARED((NP2, HALF), _f32),
          pltpu.VMEM((2 * NB, 2, CHUNK), jnp.int32),
          pltpu.VMEM((NB, CHUNK, HALF), _f32),
      ] + [pltpu.SemaphoreType.DMA] * (3 * NB),
      compiler_params=pltpu.CompilerParams(use_tc_tiling_on_sc=False),
  )(ei3, hp3)


def _prologue_body(deg_ref, x_ref, w_ref, dis_ref, hp_ref):
  d = deg_ref[0, :, 0:1] + deg_ref[1, :, 0:1] + 1.0
  dis = lax.rsqrt(d)
  h = jnp.dot(x_ref[...], w_ref[...], preferred_element_type=_f32)
  hp = h * dis
  dis_ref[...] = dis
  hp_ref[0] = hp[:, :HALF]
  hp_ref[1] = hp[:, HALF:]


def _prologue_call(deg2, x, w_in):
  return pl.pallas_call(
      _prologue_body,
      grid=(GRID,),
      in_specs=[
          pl.BlockSpec((NCORE, RBLK, 8), lambda i: (0, i, 0)),
          pl.BlockSpec((RBLK, D_IN), lambda i: (i, 0)),
          pl.BlockSpec((D_IN, D), lambda i: (0, 0)),
      ],
      out_specs=[
          pl.BlockSpec((RBLK, 1), lambda i: (i, 0)),
          pl.BlockSpec((NCORE, RBLK, HALF), lambda i: (0, i, 0)),
      ],
      out_shape=[
          jax.ShapeDtypeStruct((NP2, 1), _f32),
          jax.ShapeDtypeStruct((NCORE, NP2, HALF), _f32),
      ],
  )(deg2, x, w_in)


def _layer_body(residual, out_h, acc_ref, dis_ref, b_ref, g_ref, be_ref,
                wn_ref, *rest):
  if residual:
    hprev_ref = rest[0]
    rest = rest[1:]
  if out_h:
    h_ref = rest[0]
    hp_ref = rest[1]
  else:
    hp_ref = rest[0]
  t = jnp.concatenate([acc_ref[0], acc_ref[1]], axis=1)
  t = t * dis_ref[...] + b_ref[...]
  mu = jnp.mean(t, axis=1, keepdims=True)
  var = jnp.mean(jnp.square(t - mu), axis=1, keepdims=True)
  t = (t - mu) * lax.rsqrt(var + 1e-5) * g_ref[...] + be_ref[...]
  t = jnp.maximum(t, 0.0)
  if residual:
    t = t + hprev_ref[...]
  if out_h:
    h_ref[...] = t
  z = jnp.dot(t, wn_ref[...], preferred_element_type=_f32) * dis_ref[...]
  hp_ref[0] = z[:, :HALF]
  hp_ref[1] = z[:, HALF:]


def _layer_call(acc, dis, bvec, gvec, bevec, wn, hprev=None, out_h=False):
  residual = hprev is not None
  body = functools.partial(_layer_body, residual, out_h)
  in_specs = [
      pl.BlockSpec((NCORE, RBLK, HALF), lambda i: (0, i, 0)),
      pl.BlockSpec((RBLK, 1), lambda i: (i, 0)),
      pl.BlockSpec((1, D), lambda i: (0, 0)),
      pl.BlockSpec((1, D), lambda i: (0, 0)),
      pl.BlockSpec((1, D), lambda i: (0, 0)),
      pl.BlockSpec((D, D), lambda i: (0, 0)),
  ]
  args = [acc, dis, bvec, gvec, bevec, wn]
  if residual:
    in_specs.append(pl.BlockSpec((RBLK, D), lambda i: (i, 0)))
    args.append(hprev)
  out_specs = []
  out_shape = []
  if out_h:
    out_specs.append(pl.BlockSpec((RBLK, D), lambda i: (i, 0)))
    out_shape.append(jax.ShapeDtypeStruct((NP2, D), _f32))
  out_specs.append(pl.BlockSpec((NCORE, RBLK, HALF), lambda i: (0, i, 0)))
  out_shape.append(jax.ShapeDtypeStruct((NCORE, NP2, HALF), _f32))
  return pl.pallas_call(
      body,
      grid=(GRID,),
      in_specs=in_specs,
      out_specs=out_specs,
      out_shape=out_shape,
  )(*args)


def _pool_body(acc_ref, dis_ref, b_ref, g_ref, be_ref, batch_ref, ones_ref,
               psum_ref, cnt_ref):
  t = jnp.concatenate([acc_ref[0], acc_ref[1]], axis=1)
  t = t * dis_ref[...] + b_ref[...]
  mu = jnp.mean(t, axis=1, keepdims=True)
  var = jnp.mean(jnp.square(t - mu), axis=1, keepdims=True)
  t = (t - mu) * lax.rsqrt(var + 1e-5) * g_ref[...] + be_ref[...]
  t = jnp.maximum(t, 0.0)
  gids = lax.broadcasted_iota(jnp.int32, (1, G), 1)
  bh = (batch_ref[...] == gids).astype(_f32)
  dn = (((0,), (0,)), ((), ()))
  ps = lax.dot_general(bh, t, dn, preferred_element_type=_f32)
  cn = lax.dot_general(bh, ones_ref[...], dn, preferred_element_type=_f32)
  i = pl.program_id(0)

  @pl.when(i == 0)
  def _():
    psum_ref[...] = ps
    cnt_ref[...] = cn

  @pl.when(i > 0)
  def _():
    psum_ref[...] += ps
    cnt_ref[...] += cn


def _pool_call(acc, dis, bvec, gvec, bevec, batch2d, ones_col):
  return pl.pallas_call(
      _pool_body,
      grid=(GRID,),
      in_specs=[
          pl.BlockSpec((NCORE, RBLK, HALF), lambda i: (0, i, 0)),
          pl.BlockSpec((RBLK, 1), lambda i: (i, 0)),
          pl.BlockSpec((1, D), lambda i: (0, 0)),
          pl.BlockSpec((1, D), lambda i: (0, 0)),
          pl.BlockSpec((1, D), lambda i: (0, 0)),
          pl.BlockSpec((RBLK, 1), lambda i: (i, 0)),
          pl.BlockSpec((RBLK, 1), lambda i: (i, 0)),
      ],
      out_specs=[
          pl.BlockSpec((G, D), lambda i: (0, 0)),
          pl.BlockSpec((G, 1), lambda i: (0, 0)),
      ],
      out_shape=[
          jax.ShapeDtypeStruct((G, D), _f32),
          jax.ShapeDtypeStruct((G, 1), _f32),
      ],
  )(acc, dis, bvec, gvec, bevec, batch2d, ones_col)


def _head_body(ps_ref, cnt_ref, w1_ref, b1_ref, w2_ref, b2_ref, out_ref):
  pooled = ps_ref[...] / jnp.maximum(cnt_ref[...], 1.0)
  a = jnp.dot(pooled, w1_ref[...], preferred_element_type=_f32) + b1_ref[...]
  a = jnp.maximum(a, 0.0)
  o = jnp.dot(a, w2_ref[...], preferred_element_type=_f32) + b2_ref[...]
  out_ref[...] = jax.nn.sigmoid(o)


def _head_call(psum, cnt, w1, b1, w2, b2):
  return pl.pallas_call(
      _head_body,
      out_shape=jax.ShapeDtypeStruct((G, 1), _f32),
  )(psum, cnt, w1, b1, w2, b2)


def kernel(x, edge_index, batch, W_in, b_in, g_in, be_in, W_mid, b_mid,
           g_mid, be_mid, W_out, b_out, g_out, be_out, W1, b1, W2, b2):
  src = edge_index[0]
  dst = edge_index[1]
  epad = EP - E
  src2d = jnp.concatenate(
      [src, jnp.zeros((epad,), jnp.int32)]).reshape(NROW, CHUNK)
  dst2d = jnp.concatenate(
      [dst, jnp.full((epad,), N, jnp.int32)]).reshape(NROW, CHUNK)
  ei3 = jnp.stack([src2d, dst2d], axis=1)
  xp = jnp.concatenate([x, jnp.zeros((NP2 - N, D_IN), _f32)])

  zeros8 = jnp.zeros((NP2, 8), _f32)
  ones8 = jnp.ones((CHUNK, 8), _f32)
  deg2 = _deg_call(dst2d, zeros8, ones8)

  dis, hp = _prologue_call(deg2, xp, W_in)

  params = ([(b_in, g_in, be_in)] + [(b_mid, g_mid, be_mid)] * 6
            + [(b_out, g_out, be_out)])
  params = [(b.reshape(1, D), g.reshape(1, D), be.reshape(1, D))
            for (b, g, be) in params]

  h_prev = None
  for i in range(8):
    acc = _agg_call(ei3, hp)
    bv, gv, bev = params[i]
    if i < 7:
      wn = W_mid if i < 6 else W_out
      out_h = i in (1, 3, 5)
      hprev = h_prev if i in (2, 4, 6) else None
      res = _layer_call(acc, dis, bv, gv, bev, wn, hprev=hprev, out_h=out_h)
      if out_h:
        h_prev, hp = res
      else:
        hp = res[0]
    else:
      batch2d = jnp.concatenate(
          [batch, jnp.full((NP2 - N,), G, jnp.int32)]).reshape(NP2, 1)
      ones_col = jnp.ones((NP2, 1), _f32)
      psum, cnt = _pool_call(acc, dis, bv, gv, bev, batch2d, ones_col)

  return _head_call(psum, cnt, W1, b1.reshape(1, D), W2, b2.reshape(1, 1))
